# Initial kernel scaffold; baseline (speedup 1.0000x reference)
import jax
import jax.numpy as jnp
from jax.experimental import pallas as pl


def kernel(X_H, X_G, hg_pairs, g_edge_index, W_h1, b_h1, W_h2, b_h2, W_g1, b_g1, W_g2, b_g2, Wa, ba, Wb, bb, Wc, bc, Wo, bo, ln1_g, ln1_b, ln2_g, ln2_b, Wf, bf):
    raise NotImplementedError("write your pallas kernel here")


# speedup vs baseline: 5.2989x; 5.2989x over previous
import functools

import jax
import jax.numpy as jnp
from jax import lax
from jax.experimental import pallas as pl
from jax.experimental.pallas import tpu as pltpu
from jax.experimental.pallas import tpu_sc as plsc

N = 10000
E = 320000
D = 128
F = 64
NCLS = 10
NPAD = 10240
PADI = 10000
NCORE = 2
NSUB = 16
NW = NCORE * NSUB
CHUNK = 128
NCHUNK = -(-E // (NW * CHUNK))
EPAD = NW * CHUNK * NCHUNK
RPT = NPAD // NSUB
BLK = 256
NB = NPAD // BLK
HI = lax.Precision.HIGHEST
f32 = jnp.float32

_MESH = plsc.VectorSubcoreMesh(core_axis_name="c", subcore_axis_name="s")


@functools.partial(
    pl.kernel,
    out_type=jax.ShapeDtypeStruct((NCORE, NPAD, F), f32),
    mesh=_MESH,
    compiler_params=pltpu.CompilerParams(use_tc_tiling_on_sc=False),
    scratch_types=[
        pltpu.VMEM((NCHUNK, CHUNK), jnp.int32),
        pltpu.VMEM((NCHUNK, CHUNK), jnp.int32),
        pltpu.VMEM((CHUNK, F), f32),
        pltpu.VMEM((16, F), f32),
        pltpu.VMEM_SHARED((NPAD, F), f32),
        pltpu.SemaphoreType.DMA,
    ],
)
def _sc_edge_pass(tab, gidx, sidx, out, gi, si, buf, zbuf, acc, sem):
    cid = lax.axis_index("c")
    sid = lax.axis_index("s")
    wid = cid * NSUB + sid
    pltpu.sync_copy(gidx.at[wid], gi)
    pltpu.sync_copy(sidx.at[wid], si)
    zv = jnp.zeros((16,), f32)
    for r in range(16):
        for c in range(F // 16):
            zbuf[r, pl.ds(c * 16, 16)] = zv
    base = sid * RPT

    @pl.loop(0, RPT // 16)
    def _zero(k):
        pltpu.sync_copy(zbuf, acc.at[pl.ds(base + k * 16, 16)])

    plsc.subcore_barrier()

    @pl.loop(0, NCHUNK)
    def _edges(j):
        pltpu.async_copy(tab.at[gi.at[j]], buf, sem).wait()
        pltpu.sync_copy(buf, acc.at[si.at[j]], add=True)

    plsc.subcore_barrier()
    pltpu.sync_copy(acc.at[pl.ds(base, RPT)], out.at[cid, pl.ds(base, RPT)])


@functools.partial(
    pl.kernel,
    out_type=jax.ShapeDtypeStruct((3, NCORE, NPAD), f32),
    mesh=_MESH,
    compiler_params=pltpu.CompilerParams(use_tc_tiling_on_sc=False),
    scratch_types=[
        pltpu.VMEM((NCHUNK, CHUNK), jnp.int32),
        pltpu.VMEM((CHUNK,), f32),
        pltpu.VMEM((16,), f32),
        pltpu.VMEM_SHARED((NPAD,), f32),
        pltpu.VMEM_SHARED((NPAD,), f32),
        pltpu.VMEM_SHARED((NPAD,), f32),
    ],
)
def _sc_degrees(vix, eix, dix, out, iv, ones, zrow, t0, t1, t2):
    cid = lax.axis_index("c")
    sid = lax.axis_index("s")
    wid = cid * NSUB + sid
    zrow[...] = jnp.zeros((16,), f32)
    for c in range(CHUNK // 16):
        ones[pl.ds(c * 16, 16)] = jnp.ones((16,), f32)
    base = sid * RPT
    for t in (t0, t1, t2):
        @pl.loop(0, RPT // 16)
        def _z(k, t=t):
            pltpu.sync_copy(zrow, t.at[pl.ds(base + k * 16, 16)])
    plsc.subcore_barrier()
    for slab, t in ((vix, t0), (eix, t1), (dix, t2)):
        pltpu.sync_copy(slab.at[wid], iv)

        @pl.loop(0, NCHUNK)
        def _s(j, t=t):
            pltpu.sync_copy(ones, t.at[iv.at[j]], add=True)
    plsc.subcore_barrier()
    for k, t in enumerate((t0, t1, t2)):
        pltpu.sync_copy(t.at[pl.ds(base, RPT)], out.at[k, cid, pl.ds(base, RPT)])


def _scales(dblk, row0):
    rows = dblk.shape[0]
    rid = row0 + lax.broadcasted_iota(jnp.int32, (rows, 1), 0)
    m = (rid < N).astype(f32)
    dv = dblk[:, 0:1] + dblk[:, 1:2]
    de = dblk[:, 2:3] + dblk[:, 3:4]
    dg = dblk[:, 4:5] + dblk[:, 5:6]
    dvi = jnp.where(dv > 0, 1.0, 0.0) * lax.rsqrt(jnp.maximum(dv, 1.0)) * m
    dei = jnp.where(de > 0, 1.0, 0.0) / jnp.maximum(de, 1.0) * m
    dinv = lax.rsqrt(1.0 + dg) * m
    return dvi, dei, dinv


def _combine_weights(W_h1, W_h2, b_h1, W_g1, W_g2, b_g1):
    def body(wh1, wh2, bh1, wg1, wg2, bg1, w12h, bh, w12g, bg):
        w12h[...] = jnp.dot(wh1[...], wh2[...], precision=HI)
        bh[...] = jnp.dot(bh1[...], wh2[...], precision=HI)
        w12g[...] = jnp.dot(wg1[...], wg2[...], precision=HI)
        bg[...] = jnp.dot(bg1[...], wg2[...], precision=HI)

    return pl.pallas_call(
        body,
        out_shape=[
            jax.ShapeDtypeStruct((D, F), f32),
            jax.ShapeDtypeStruct((1, F), f32),
            jax.ShapeDtypeStruct((D, F), f32),
            jax.ShapeDtypeStruct((1, F), f32),
        ],
    )(W_h1, W_h2, b_h1.reshape(1, -1), W_g1, W_g2, b_g1.reshape(1, -1))


def _project(XH, XG, w12h, bh, w12g, bg, degt):
    def body(xh, xg, wh, bh_, wg, bg_, dg, mh, mg):
        i = pl.program_id(0)
        dvi, _, dinv = _scales(dg[...], i * BLK)
        mh[...] = (jnp.dot(xh[...], wh[...], precision=HI) + bh_[...]) * dvi
        mg[...] = (jnp.dot(xg[...], wg[...], precision=HI) + bg_[...]) * dinv

    return pl.pallas_call(
        body,
        grid=(NB,),
        in_specs=[
            pl.BlockSpec((BLK, D), lambda i: (i, 0)),
            pl.BlockSpec((BLK, D), lambda i: (i, 0)),
            pl.BlockSpec((D, F), lambda i: (0, 0)),
            pl.BlockSpec((1, F), lambda i: (0, 0)),
            pl.BlockSpec((D, F), lambda i: (0, 0)),
            pl.BlockSpec((1, F), lambda i: (0, 0)),
            pl.BlockSpec((BLK, 6), lambda i: (i, 0)),
        ],
        out_specs=[pl.BlockSpec((BLK, F), lambda i: (i, 0))] * 2,
        out_shape=[jax.ShapeDtypeStruct((NPAD, F), f32)] * 2,
    )(XH, XG, w12h, bh, w12g, bg, degt)


def _merge(p, degt, sel, add=None, bias=None):
    has_add = add is not None
    has_bias = bias is not None

    def body(*refs):
        i = pl.program_id(0)
        it = iter(refs)
        pr = next(it)[...]
        dg = next(it)[...]
        a = next(it)[...] if has_add else 0.0
        b = next(it)[...] if has_bias else None
        o = next(it)
        dvi, dei, dinv = _scales(dg, i * BLK)
        s = {"dvi": dvi, "dei": dei, "dinv": dinv}[sel]
        v = (pr[0] + pr[1] + a) * s
        if has_bias:
            v = (v + b) * s
        o[...] = v

    in_specs = [
        pl.BlockSpec((NCORE, BLK, F), lambda i: (0, i, 0)),
        pl.BlockSpec((BLK, 6), lambda i: (i, 0)),
    ]
    args = [p, degt]
    if has_add:
        in_specs.append(pl.BlockSpec((BLK, F), lambda i: (i, 0)))
        args.append(add)
    if has_bias:
        in_specs.append(pl.BlockSpec((1, F), lambda i: (0, 0)))
        args.append(bias.reshape(1, -1))
    return pl.pallas_call(
        body,
        grid=(NB,),
        in_specs=in_specs,
        out_specs=pl.BlockSpec((BLK, F), lambda i: (i, 0)),
        out_shape=jax.ShapeDtypeStruct((NPAD, F), f32),
    )(*args)


ABLK = 128
NAB = NPAD // ABLK


def _attn(h, g, Wa, ba, Wb, bb, Wc, bc):
    def body(hr, gr, wa, ba_, wb, bb_, wc, bc_,
             sh_o, fh_o, mzh_o, fg_o, mh, zh, fh, mg_, zg, fg):
        i = pl.program_id(0)

        @pl.when(i == 0)
        def _init():
            mh[...] = jnp.full((1, 1), -1e30, f32)
            zh[...] = jnp.zeros((1, 1), f32)
            fh[...] = jnp.zeros((1, F), f32)
            mg_[...] = jnp.full((1, 1), -1e30, f32)
            zg[...] = jnp.zeros((1, 1), f32)
            fg[...] = jnp.zeros((1, F), f32)

        rid = i * ABLK + lax.broadcasted_iota(jnp.int32, (ABLK, 1), 0)
        mask = rid < N

        def branch(x, m_ref, z_ref, f_ref, s_out):
            a = jnp.tanh(jnp.dot(x, wa[...], precision=HI) + ba_[...])
            bg = jnp.dot(x, wb[...], precision=HI) + bb_[...]
            bg = 1.0 / (1.0 + jnp.exp(-bg))
            s = jnp.dot(a * bg, wc[...], precision=HI) + bc_[...]
            s = jnp.where(mask, s, -1e30)
            if s_out is not None:
                s_out[...] = s
            m_old = m_ref[0, 0]
            z_old = z_ref[0, 0]
            m_new = jnp.maximum(m_old, jnp.max(s))
            corr = jnp.exp(m_old - m_new)
            e = jnp.exp(s - m_new)
            z_new = z_old * corr + jnp.sum(e)
            f_new = f_ref[...] * corr + jnp.sum(e * x, axis=0, keepdims=True)
            m_ref[...] = jnp.full((1, 1), m_new, f32)
            z_ref[...] = jnp.full((1, 1), z_new, f32)
            f_ref[...] = f_new
            return m_new, z_new, f_new

        mhv, zhv, fhv = branch(hr[...], mh, zh, fh, sh_o)
        _, zgv, fgv = branch(gr[...], mg_, zg, fg, None)
        fh_o[...] = fhv / zhv
        mzh_o[...] = jnp.concatenate(
            [jnp.full((1, 1), mhv, f32), jnp.full((1, 1), zhv, f32)], axis=1)
        fg_o[...] = fgv / zgv

    return pl.pallas_call(
        body,
        grid=(NAB,),
        in_specs=[
            pl.BlockSpec((ABLK, F), lambda i: (i, 0)),
            pl.BlockSpec((ABLK, F), lambda i: (i, 0)),
            pl.BlockSpec((F, 256), lambda i: (0, 0)),
            pl.BlockSpec((1, 256), lambda i: (0, 0)),
            pl.BlockSpec((F, 256), lambda i: (0, 0)),
            pl.BlockSpec((1, 256), lambda i: (0, 0)),
            pl.BlockSpec((256, 1), lambda i: (0, 0)),
            pl.BlockSpec((1, 1), lambda i: (0, 0)),
        ],
        out_specs=[
            pl.BlockSpec((ABLK, 1), lambda i: (i, 0)),
            pl.BlockSpec((1, F), lambda i: (0, 0)),
            pl.BlockSpec((1, 2), lambda i: (0, 0)),
            pl.BlockSpec((1, F), lambda i: (0, 0)),
        ],
        out_shape=[
            jax.ShapeDtypeStruct((NPAD, 1), f32),
            jax.ShapeDtypeStruct((1, F), f32),
            jax.ShapeDtypeStruct((1, 2), f32),
            jax.ShapeDtypeStruct((1, F), f32),
        ],
        scratch_shapes=[pltpu.VMEM((1, 1), f32), pltpu.VMEM((1, 1), f32),
                        pltpu.VMEM((1, F), f32), pltpu.VMEM((1, 1), f32),
                        pltpu.VMEM((1, 1), f32), pltpu.VMEM((1, F), f32)],
    )(h, g, Wa, ba.reshape(1, -1), Wb, bb.reshape(1, -1), Wc, bc.reshape(1, -1))


def _head(s_h, mz, feat_h, feat_g, Wo, bo, l1g, l1b, l2g, l2b, Wf, bf):
    def body(s, mzr, fh, fg, wo, bo_, g1, b1, g2, b2, wf, bf_, lo, ws):
        m = mzr[0, 0]
        z = mzr[0, 1]
        ws[...] = jnp.exp(s[...] - m) / z

        def ln(x, gg, bb_):
            mu = jnp.mean(x, axis=-1, keepdims=True)
            va = jnp.mean((x - mu) ** 2, axis=-1, keepdims=True)
            return (x - mu) * lax.rsqrt(va + 1e-5) * gg + bb_

        ha = ln(jnp.dot(fh[...], wo[...], precision=HI) + bo_[...], g1[...], b1[...])
        ga = ln(jnp.dot(fg[...], wo[...], precision=HI) + bo_[...], g1[...], b1[...])
        xc = ln(jnp.concatenate([ha, ga], axis=1), g2[...], b2[...])
        lo[...] = jnp.dot(xc, wf[...], precision=HI) + bf_[...]

    return pl.pallas_call(
        body,
        grid=(NB,),
        in_specs=[
            pl.BlockSpec((BLK, 1), lambda i: (i, 0)),
            pl.BlockSpec((1, 2), lambda i: (0, 0)),
            pl.BlockSpec((1, F), lambda i: (0, 0)),
            pl.BlockSpec((1, F), lambda i: (0, 0)),
            pl.BlockSpec((F, F), lambda i: (0, 0)),
            pl.BlockSpec((1, F), lambda i: (0, 0)),
            pl.BlockSpec((1, F), lambda i: (0, 0)),
            pl.BlockSpec((1, F), lambda i: (0, 0)),
            pl.BlockSpec((1, 2 * F), lambda i: (0, 0)),
            pl.BlockSpec((1, 2 * F), lambda i: (0, 0)),
            pl.BlockSpec((2 * F, NCLS), lambda i: (0, 0)),
            pl.BlockSpec((1, NCLS), lambda i: (0, 0)),
        ],
        out_specs=[
            pl.BlockSpec((1, NCLS), lambda i: (0, 0)),
            pl.BlockSpec((BLK, 1), lambda i: (i, 0)),
        ],
        out_shape=[
            jax.ShapeDtypeStruct((1, NCLS), f32),
            jax.ShapeDtypeStruct((NPAD, 1), f32),
        ],
    )(s_h, mz, feat_h, feat_g, Wo, bo.reshape(1, -1), l1g.reshape(1, -1),
      l1b.reshape(1, -1), l2g.reshape(1, -1), l2b.reshape(1, -1), Wf,
      bf.reshape(1, -1))


def kernel(X_H, X_G, hg_pairs, g_edge_index, W_h1, b_h1, W_h2, b_h2,
           W_g1, b_g1, W_g2, b_g2, Wa, ba, Wb, bb, Wc, bc, Wo, bo,
           ln1_g, ln1_b, ln2_g, ln2_b, Wf, bf):
    def slab(ix):
        pad = jnp.full((EPAD - E,), PADI, jnp.int32)
        return jnp.concatenate([ix, pad]).reshape(NW, NCHUNK, CHUNK)

    v_s = slab(hg_pairs[0])
    e_s = slab(hg_pairs[1])
    src_s = slab(g_edge_index[0])
    dst_s = slab(g_edge_index[1])
    zrows = jnp.zeros((NPAD - N, D), f32)
    XHp = jnp.concatenate([X_H, zrows], axis=0)
    XGp = jnp.concatenate([X_G, zrows], axis=0)

    deg = _sc_degrees(v_s, e_s, dst_s)
    degt = jnp.transpose(deg.reshape(6, NPAD))
    w12h, bh, w12g, bg = _combine_weights(W_h1, W_h2, b_h1, W_g1, W_g2, b_g1)
    Mh, Mg = _project(XHp, XGp, w12h, bh, w12g, bg, degt)

    p = _sc_edge_pass(Mh, v_s, e_s)
    xe = _merge(p, degt, "dei")
    p = _sc_edge_pass(xe, e_s, v_s)
    in2 = _merge(p, degt, "dvi", bias=b_h2)
    p = _sc_edge_pass(in2, v_s, e_s)
    xe2 = _merge(p, degt, "dei")
    p = _sc_edge_pass(xe2, e_s, v_s)
    h = _merge(p, degt, "dvi")

    q = _sc_edge_pass(Mg, src_s, dst_s)
    in2g = _merge(q, degt, "dinv", add=Mg, bias=b_g2)
    q = _sc_edge_pass(in2g, src_s, dst_s)
    g = _merge(q, degt, "dinv", add=in2g)

    s_h, feat_h, mz, feat_g = _attn(h, g, Wa, ba, Wb, bb, Wc, bc)
    logits, ws = _head(s_h, mz, feat_h, feat_g, Wo, bo,
                       ln1_g, ln1_b, ln2_g, ln2_b, Wf, bf)
    return logits, ws[:N, 0]
